# baseline (speedup 1.0000x reference)
import jax
import jax.numpy as jnp
from jax import lax
from jax.experimental import pallas as pl
from jax.experimental.pallas import tpu as pltpu
from jax.experimental.pallas import tpu_sc as plsc

VOCAB = 1000000
EMBED = 64
MAX_LEN = 200
BATCH = 4096
SEQ_LEN = 200

NUM_WORKERS = 32

SLAB = 256
NFULL = (VOCAB // SLAB)
T1_STEPS = (NFULL + NUM_WORKERS - 1) // NUM_WORKERS

QB = 128
NQ = BATCH // QB
UNITS = SEQ_LEN * NQ
UNITS_PER_W = UNITS // NUM_WORKERS


TBLK = 512
TGRID = (VOCAB + TBLK - 1) // TBLK


def _tc_transpose_body(x_ref, o_ref):
    xt = x_ref[...].T
    o_ref[...] = jnp.concatenate([xt, jnp.zeros_like(xt)], axis=1)


def _gather_body(seqT_hbm, table_hbm, pos_hbm, out_hbm,
                 idx_all, pos_v, g0, g1, p0, p1,
                 gs0, gs1, ws0, ws1):
    gbuf = (g0, g1)
    pbuf = (p0, p1)
    gsem = (gs0, gs1)
    wsem = (ws0, ws1)
    wid = lax.axis_index("s") * 2 + lax.axis_index("c")
    ubase = wid * UNITS_PER_W

    pltpu.sync_copy(seqT_hbm.at[pl.ds(wid * UNITS_PER_W, UNITS_PER_W)],
                    idx_all)
    pltpu.sync_copy(pos_hbm, pos_v)

    iotav = lax.iota(jnp.int32, 16)
    ehj = [(iotav + 16 * j) // 8 for j in range(4)]
    elj = [(iotav + 16 * j) % 8 for j in range(4)]

    def gather_start(uu, buf):
        pltpu.async_copy(table_hbm.at[idx_all.at[uu]], gbuf[buf], gsem[buf])

    def gather_wait(uu, buf):
        pltpu.make_async_copy(table_hbm.at[idx_all.at[uu]], gbuf[buf],
                              gsem[buf]).wait()

    def write_start(uu, buf):
        U = ubase + uu
        l = U // NQ
        q = U % NQ
        pltpu.async_copy(pbuf[buf].at[:, :, pl.ds(0, QB)],
                         out_hbm.at[l, :, q], wsem[buf])

    def write_wait(uu, buf):
        U = ubase + uu
        l = U // NQ
        q = U % NQ
        pltpu.make_async_copy(pbuf[buf].at[:, :, pl.ds(0, QB)],
                              out_hbm.at[l, :, q], wsem[buf]).wait()

    def transform(uu, buf):
        U = ubase + uu
        l = U // NQ
        G = gbuf[buf]
        P = pbuf[buf]
        posr = [pos_v[l, pl.ds(16 * j, 16)] for j in range(4)]

        @plsc.parallel_loop(0, QB, unroll=2)
        def _(t):
            t16 = jnp.full((16,), t, jnp.int32)
            for j in range(4):
                v = G[t, pl.ds(16 * j, 16)] + posr[j]
                plsc.store_scatter(P, [ehj[j], elj[j], t16], v)

    gather_start(0, 0)
    gather_start(1, 1)

    def outer(cc, _):
        for buf in range(2):
            uu = 2 * cc + buf
            gather_wait(uu, buf)

            @pl.when(uu >= 2)
            def _(uu=uu, buf=buf):
                write_wait(uu - 2, buf)

            transform(uu, buf)
            write_start(uu, buf)

            @pl.when(uu + 2 < UNITS_PER_W)
            def _(uu=uu, buf=buf):
                gather_start(uu + 2, buf)
        return 0

    lax.fori_loop(0, UNITS_PER_W // 2, outer, 0)

    write_wait(UNITS_PER_W - 2, 0)
    write_wait(UNITS_PER_W - 1, 1)


def kernel(seq, token_table, pos_table):
    mesh = plsc.VectorSubcoreMesh(core_axis_name="c", subcore_axis_name="s")

    tT = token_table.T
    t2 = pl.pallas_call(
        _tc_transpose_body,
        out_shape=jax.ShapeDtypeStruct((VOCAB, 128), jnp.float32),
        grid=(TGRID,),
        in_specs=[pl.BlockSpec((EMBED, TBLK), lambda i: (0, i))],
        out_specs=pl.BlockSpec((TBLK, 128), lambda i: (i, 0)),
    )(tT)

    seqT = seq.T.reshape(UNITS, QB)
    table = t2
    k5 = pl.kernel(
        _gather_body,
        out_type=jax.ShapeDtypeStruct((SEQ_LEN, 8, NQ, 8, QB), jnp.float32),
        mesh=mesh,
        scratch_types=[
            pltpu.VMEM((UNITS_PER_W, QB), jnp.int32),
            pltpu.VMEM((MAX_LEN, EMBED), jnp.float32),
            pltpu.VMEM((QB, 2 * EMBED), jnp.float32),
            pltpu.VMEM((QB, 2 * EMBED), jnp.float32),
            pltpu.VMEM((8, 8, QB + 1), jnp.float32),
            pltpu.VMEM((8, 8, QB + 1), jnp.float32),
            pltpu.SemaphoreType.DMA,
            pltpu.SemaphoreType.DMA,
            pltpu.SemaphoreType.DMA,
            pltpu.SemaphoreType.DMA,
        ],
        compiler_params=pltpu.CompilerParams(use_tc_tiling_on_sc=False,
                                             needs_layout_passes=False,
                                             disable_bounds_checks=True),
    )(seqT, table, pos_table)
    return k5.transpose(2, 4, 0, 1, 3).reshape(BATCH, SEQ_LEN, EMBED)


# speedup vs baseline: 2.0280x; 2.0280x over previous
import jax
import jax.numpy as jnp
from jax import lax
from jax.experimental import pallas as pl
from jax.experimental.pallas import tpu as pltpu
from jax.experimental.pallas import tpu_sc as plsc

VOCAB = 1000000
EMBED = 64
MAX_LEN = 200
BATCH = 4096
SEQ_LEN = 200

NUM_WORKERS = 32

SLAB = 256
NFULL = (VOCAB // SLAB)
T1_STEPS = (NFULL + NUM_WORKERS - 1) // NUM_WORKERS

QB = 128
NQ = BATCH // QB
UNITS = SEQ_LEN * NQ
UNITS_PER_W = UNITS // NUM_WORKERS


TBLK = 2048
TGRID = (VOCAB + TBLK - 1) // TBLK


def _tc_transpose_body(x_ref, o_ref):
    xt = x_ref[...].T
    o_ref[...] = jnp.concatenate([xt, jnp.zeros_like(xt)], axis=1)


def _gather_body(seqT_hbm, table_hbm, pos_hbm, out_hbm,
                 idx_all, pos_v, g0, g1, p0, p1,
                 gs0, gs1, ws0, ws1):
    gbuf = (g0, g1)
    pbuf = (p0, p1)
    gsem = (gs0, gs1)
    wsem = (ws0, ws1)
    wid = lax.axis_index("s") * 2 + lax.axis_index("c")
    ubase = wid * UNITS_PER_W

    pltpu.sync_copy(seqT_hbm.at[pl.ds(wid * UNITS_PER_W, UNITS_PER_W)],
                    idx_all)
    pltpu.sync_copy(pos_hbm, pos_v)

    iotav = lax.iota(jnp.int32, 16)
    ehj = [(iotav + 16 * j) // 8 for j in range(4)]
    elj = [(iotav + 16 * j) % 8 for j in range(4)]

    def gather_start(uu, buf):
        pltpu.async_copy(table_hbm.at[idx_all.at[uu]], gbuf[buf], gsem[buf])

    def gather_wait(uu, buf):
        pltpu.make_async_copy(table_hbm.at[idx_all.at[uu]], gbuf[buf],
                              gsem[buf]).wait()

    def write_start(uu, buf):
        U = ubase + uu
        l = U // NQ
        q = U % NQ
        pltpu.async_copy(pbuf[buf].at[:, :, pl.ds(0, QB)],
                         out_hbm.at[l, :, q], wsem[buf])

    def write_wait(uu, buf):
        U = ubase + uu
        l = U // NQ
        q = U % NQ
        pltpu.make_async_copy(pbuf[buf].at[:, :, pl.ds(0, QB)],
                              out_hbm.at[l, :, q], wsem[buf]).wait()

    def transform(uu, buf):
        U = ubase + uu
        l = U // NQ
        G = gbuf[buf]
        P = pbuf[buf]
        posr = [pos_v[l, pl.ds(16 * j, 16)] for j in range(4)]

        @plsc.parallel_loop(0, QB, unroll=2)
        def _(t):
            t16 = jnp.full((16,), t, jnp.int32)
            for j in range(4):
                v = G[t, pl.ds(16 * j, 16)] + posr[j]
                plsc.store_scatter(P, [ehj[j], elj[j], t16], v)

    gather_start(0, 0)
    gather_start(1, 1)

    def outer(cc, _):
        for buf in range(2):
            uu = 2 * cc + buf
            gather_wait(uu, buf)

            @pl.when(uu >= 2)
            def _(uu=uu, buf=buf):
                write_wait(uu - 2, buf)

            transform(uu, buf)
            write_start(uu, buf)

            @pl.when(uu + 2 < UNITS_PER_W)
            def _(uu=uu, buf=buf):
                gather_start(uu + 2, buf)
        return 0

    lax.fori_loop(0, UNITS_PER_W // 2, outer, 0)

    write_wait(UNITS_PER_W - 2, 0)
    write_wait(UNITS_PER_W - 1, 1)


def kernel(seq, token_table, pos_table):
    mesh = plsc.VectorSubcoreMesh(core_axis_name="c", subcore_axis_name="s")

    tT = token_table.T
    t2 = pl.pallas_call(
        _tc_transpose_body,
        out_shape=jax.ShapeDtypeStruct((VOCAB, 128), jnp.float32),
        grid=(TGRID,),
        in_specs=[pl.BlockSpec((EMBED, TBLK), lambda i: (0, i))],
        out_specs=pl.BlockSpec((TBLK, 128), lambda i: (i, 0)),
    )(tT)

    seqT = seq.T.reshape(UNITS, QB)
    table = t2
    k5 = pl.kernel(
        _gather_body,
        out_type=jax.ShapeDtypeStruct((SEQ_LEN, 8, NQ, 8, QB), jnp.float32),
        mesh=mesh,
        scratch_types=[
            pltpu.VMEM((UNITS_PER_W, QB), jnp.int32),
            pltpu.VMEM((MAX_LEN, EMBED), jnp.float32),
            pltpu.VMEM((QB, 2 * EMBED), jnp.float32),
            pltpu.VMEM((QB, 2 * EMBED), jnp.float32),
            pltpu.VMEM((8, 8, QB + 1), jnp.float32),
            pltpu.VMEM((8, 8, QB + 1), jnp.float32),
            pltpu.SemaphoreType.DMA,
            pltpu.SemaphoreType.DMA,
            pltpu.SemaphoreType.DMA,
            pltpu.SemaphoreType.DMA,
        ],
        compiler_params=pltpu.CompilerParams(use_tc_tiling_on_sc=False,
                                             needs_layout_passes=False,
                                             disable_bounds_checks=True),
    )(seqT, table, pos_table)
    return k5.transpose(2, 4, 0, 1, 3).reshape(BATCH, SEQ_LEN, EMBED)


# speedup vs baseline: 2.7833x; 1.3725x over previous
import jax
import jax.numpy as jnp
from jax import lax
from jax.experimental import pallas as pl
from jax.experimental.pallas import tpu as pltpu
from jax.experimental.pallas import tpu_sc as plsc

VOCAB = 1000000
EMBED = 64
MAX_LEN = 200
BATCH = 4096
SEQ_LEN = 200

NUM_WORKERS = 32

SLAB = 256
NFULL = (VOCAB // SLAB)
T1_STEPS = (NFULL + NUM_WORKERS - 1) // NUM_WORKERS

QB = 128
NQ = BATCH // QB
UNITS = SEQ_LEN * NQ
UNITS_PER_W = UNITS // NUM_WORKERS


TBLK = 8192
TGRID = (VOCAB + TBLK - 1) // TBLK


def _tc_transpose_body(x_ref, o_ref):
    xt = x_ref[...].T
    o_ref[...] = jnp.concatenate([xt, jnp.zeros_like(xt)], axis=1)


def _gather_body(seqT_hbm, table_hbm, pos_hbm, out_hbm,
                 idx_all, pos_v, g0, g1, p0, p1,
                 gs0, gs1, ws0, ws1):
    gbuf = (g0, g1)
    pbuf = (p0, p1)
    gsem = (gs0, gs1)
    wsem = (ws0, ws1)
    wid = lax.axis_index("s") * 2 + lax.axis_index("c")
    ubase = wid * UNITS_PER_W

    pltpu.sync_copy(seqT_hbm.at[pl.ds(wid * UNITS_PER_W, UNITS_PER_W)],
                    idx_all)
    pltpu.sync_copy(pos_hbm, pos_v)

    iotav = lax.iota(jnp.int32, 16)
    ehj = [(iotav + 16 * j) // 8 for j in range(4)]
    elj = [(iotav + 16 * j) % 8 for j in range(4)]

    def gather_start(uu, buf):
        pltpu.async_copy(table_hbm.at[idx_all.at[uu]], gbuf[buf], gsem[buf])

    def gather_wait(uu, buf):
        pltpu.make_async_copy(table_hbm.at[idx_all.at[uu]], gbuf[buf],
                              gsem[buf]).wait()

    def write_start(uu, buf):
        U = ubase + uu
        l = U // NQ
        q = U % NQ
        pltpu.async_copy(pbuf[buf].at[:, :, pl.ds(0, QB)],
                         out_hbm.at[l, :, q], wsem[buf])

    def write_wait(uu, buf):
        U = ubase + uu
        l = U // NQ
        q = U % NQ
        pltpu.make_async_copy(pbuf[buf].at[:, :, pl.ds(0, QB)],
                              out_hbm.at[l, :, q], wsem[buf]).wait()

    def transform(uu, buf):
        U = ubase + uu
        l = U // NQ
        G = gbuf[buf]
        P = pbuf[buf]
        posr = [pos_v[l, pl.ds(16 * j, 16)] for j in range(4)]

        @plsc.parallel_loop(0, QB, unroll=2)
        def _(t):
            t16 = jnp.full((16,), t, jnp.int32)
            for j in range(4):
                v = G[t, pl.ds(16 * j, 16)] + posr[j]
                plsc.store_scatter(P, [ehj[j], elj[j], t16], v)

    gather_start(0, 0)
    gather_start(1, 1)

    def outer(cc, _):
        for buf in range(2):
            uu = 2 * cc + buf
            gather_wait(uu, buf)

            @pl.when(uu >= 2)
            def _(uu=uu, buf=buf):
                write_wait(uu - 2, buf)

            transform(uu, buf)
            write_start(uu, buf)

            @pl.when(uu + 2 < UNITS_PER_W)
            def _(uu=uu, buf=buf):
                gather_start(uu + 2, buf)
        return 0

    lax.fori_loop(0, UNITS_PER_W // 2, outer, 0)

    write_wait(UNITS_PER_W - 2, 0)
    write_wait(UNITS_PER_W - 1, 1)


def kernel(seq, token_table, pos_table):
    mesh = plsc.VectorSubcoreMesh(core_axis_name="c", subcore_axis_name="s")

    tT = token_table.T
    t2 = pl.pallas_call(
        _tc_transpose_body,
        out_shape=jax.ShapeDtypeStruct((VOCAB, 128), jnp.float32),
        grid=(TGRID,),
        in_specs=[pl.BlockSpec((EMBED, TBLK), lambda i: (0, i))],
        out_specs=pl.BlockSpec((TBLK, 128), lambda i: (i, 0)),
    )(tT)

    seqT = seq.T.reshape(UNITS, QB)
    table = t2
    k5 = pl.kernel(
        _gather_body,
        out_type=jax.ShapeDtypeStruct((SEQ_LEN, 8, NQ, 8, QB), jnp.float32),
        mesh=mesh,
        scratch_types=[
            pltpu.VMEM((UNITS_PER_W, QB), jnp.int32),
            pltpu.VMEM((MAX_LEN, EMBED), jnp.float32),
            pltpu.VMEM((QB, 2 * EMBED), jnp.float32),
            pltpu.VMEM((QB, 2 * EMBED), jnp.float32),
            pltpu.VMEM((8, 8, QB + 1), jnp.float32),
            pltpu.VMEM((8, 8, QB + 1), jnp.float32),
            pltpu.SemaphoreType.DMA,
            pltpu.SemaphoreType.DMA,
            pltpu.SemaphoreType.DMA,
            pltpu.SemaphoreType.DMA,
        ],
        compiler_params=pltpu.CompilerParams(use_tc_tiling_on_sc=False,
                                             needs_layout_passes=False,
                                             disable_bounds_checks=True),
    )(seqT, table, pos_table)
    return k5.transpose(2, 4, 0, 1, 3).reshape(BATCH, SEQ_LEN, EMBED)


# speedup vs baseline: 2.8961x; 1.0405x over previous
import jax
import jax.numpy as jnp
from jax import lax
from jax.experimental import pallas as pl
from jax.experimental.pallas import tpu as pltpu
from jax.experimental.pallas import tpu_sc as plsc

VOCAB = 1000000
EMBED = 64
MAX_LEN = 200
BATCH = 4096
SEQ_LEN = 200

NUM_WORKERS = 32

SLAB = 256
NFULL = (VOCAB // SLAB)
T1_STEPS = (NFULL + NUM_WORKERS - 1) // NUM_WORKERS

QB = 128
NQ = BATCH // QB
UNITS = SEQ_LEN * NQ
UNITS_PER_W = UNITS // NUM_WORKERS


TBLK = 16384
TGRID = (VOCAB + TBLK - 1) // TBLK


def _tc_transpose_body(x_ref, o_ref):
    xt = x_ref[...].T
    o_ref[...] = jnp.concatenate([xt, jnp.zeros_like(xt)], axis=1)


def _gather_body(seqT_hbm, table_hbm, pos_hbm, out_hbm,
                 idx_all, pos_v, g0, g1, p0, p1,
                 gs0, gs1, ws0, ws1):
    gbuf = (g0, g1)
    pbuf = (p0, p1)
    gsem = (gs0, gs1)
    wsem = (ws0, ws1)
    wid = lax.axis_index("s") * 2 + lax.axis_index("c")
    ubase = wid * UNITS_PER_W

    pltpu.sync_copy(seqT_hbm.at[pl.ds(wid * UNITS_PER_W, UNITS_PER_W)],
                    idx_all)
    pltpu.sync_copy(pos_hbm, pos_v)

    iotav = lax.iota(jnp.int32, 16)
    ehj = [(iotav + 16 * j) // 8 for j in range(4)]
    elj = [(iotav + 16 * j) % 8 for j in range(4)]

    def gather_start(uu, buf):
        pltpu.async_copy(table_hbm.at[idx_all.at[uu]], gbuf[buf], gsem[buf])

    def gather_wait(uu, buf):
        pltpu.make_async_copy(table_hbm.at[idx_all.at[uu]], gbuf[buf],
                              gsem[buf]).wait()

    def write_start(uu, buf):
        U = ubase + uu
        l = U // NQ
        q = U % NQ
        pltpu.async_copy(pbuf[buf].at[:, :, pl.ds(0, QB)],
                         out_hbm.at[l, :, q], wsem[buf])

    def write_wait(uu, buf):
        U = ubase + uu
        l = U // NQ
        q = U % NQ
        pltpu.make_async_copy(pbuf[buf].at[:, :, pl.ds(0, QB)],
                              out_hbm.at[l, :, q], wsem[buf]).wait()

    def transform(uu, buf):
        U = ubase + uu
        l = U // NQ
        G = gbuf[buf]
        P = pbuf[buf]
        posr = [pos_v[l, pl.ds(16 * j, 16)] for j in range(4)]

        @plsc.parallel_loop(0, QB, unroll=2)
        def _(t):
            t16 = jnp.full((16,), t, jnp.int32)
            for j in range(4):
                v = G[t, pl.ds(16 * j, 16)] + posr[j]
                plsc.store_scatter(P, [ehj[j], elj[j], t16], v)

    gather_start(0, 0)
    gather_start(1, 1)

    def outer(cc, _):
        for buf in range(2):
            uu = 2 * cc + buf
            gather_wait(uu, buf)

            @pl.when(uu >= 2)
            def _(uu=uu, buf=buf):
                write_wait(uu - 2, buf)

            transform(uu, buf)
            write_start(uu, buf)

            @pl.when(uu + 2 < UNITS_PER_W)
            def _(uu=uu, buf=buf):
                gather_start(uu + 2, buf)
        return 0

    lax.fori_loop(0, UNITS_PER_W // 2, outer, 0)

    write_wait(UNITS_PER_W - 2, 0)
    write_wait(UNITS_PER_W - 1, 1)


def kernel(seq, token_table, pos_table):
    mesh = plsc.VectorSubcoreMesh(core_axis_name="c", subcore_axis_name="s")

    tT = token_table.T
    t2 = pl.pallas_call(
        _tc_transpose_body,
        out_shape=jax.ShapeDtypeStruct((VOCAB, 128), jnp.float32),
        grid=(TGRID,),
        in_specs=[pl.BlockSpec((EMBED, TBLK), lambda i: (0, i))],
        out_specs=pl.BlockSpec((TBLK, 128), lambda i: (i, 0)),
    )(tT)

    seqT = seq.T.reshape(UNITS, QB)
    table = t2
    k5 = pl.kernel(
        _gather_body,
        out_type=jax.ShapeDtypeStruct((SEQ_LEN, 8, NQ, 8, QB), jnp.float32),
        mesh=mesh,
        scratch_types=[
            pltpu.VMEM((UNITS_PER_W, QB), jnp.int32),
            pltpu.VMEM((MAX_LEN, EMBED), jnp.float32),
            pltpu.VMEM((QB, 2 * EMBED), jnp.float32),
            pltpu.VMEM((QB, 2 * EMBED), jnp.float32),
            pltpu.VMEM((8, 8, QB + 1), jnp.float32),
            pltpu.VMEM((8, 8, QB + 1), jnp.float32),
            pltpu.SemaphoreType.DMA,
            pltpu.SemaphoreType.DMA,
            pltpu.SemaphoreType.DMA,
            pltpu.SemaphoreType.DMA,
        ],
        compiler_params=pltpu.CompilerParams(use_tc_tiling_on_sc=False,
                                             needs_layout_passes=False,
                                             disable_bounds_checks=True),
    )(seqT, table, pos_table)
    return k5.transpose(2, 4, 0, 1, 3).reshape(BATCH, SEQ_LEN, EMBED)


# speedup vs baseline: 2.9301x; 1.0117x over previous
import jax
import jax.numpy as jnp
from jax import lax
from jax.experimental import pallas as pl
from jax.experimental.pallas import tpu as pltpu
from jax.experimental.pallas import tpu_sc as plsc

VOCAB = 1000000
EMBED = 64
MAX_LEN = 200
BATCH = 4096
SEQ_LEN = 200

NUM_WORKERS = 32

SLAB = 256
NFULL = (VOCAB // SLAB)
T1_STEPS = (NFULL + NUM_WORKERS - 1) // NUM_WORKERS

QB = 128
NQ = BATCH // QB
UNITS = SEQ_LEN * NQ
UNITS_PER_W = UNITS // NUM_WORKERS


TBLK = 32768
TGRID = (VOCAB + TBLK - 1) // TBLK


def _tc_transpose_body(x_ref, o_ref):
    xt = x_ref[...].T
    o_ref[...] = jnp.concatenate([xt, jnp.zeros_like(xt)], axis=1)


def _gather_body(seqT_hbm, table_hbm, pos_hbm, out_hbm,
                 idx_all, pos_v, g0, g1, p0, p1,
                 gs0, gs1, ws0, ws1):
    gbuf = (g0, g1)
    pbuf = (p0, p1)
    gsem = (gs0, gs1)
    wsem = (ws0, ws1)
    wid = lax.axis_index("s") * 2 + lax.axis_index("c")
    ubase = wid * UNITS_PER_W

    pltpu.sync_copy(seqT_hbm.at[pl.ds(wid * UNITS_PER_W, UNITS_PER_W)],
                    idx_all)
    pltpu.sync_copy(pos_hbm, pos_v)

    iotav = lax.iota(jnp.int32, 16)
    ehj = [(iotav + 16 * j) // 8 for j in range(4)]
    elj = [(iotav + 16 * j) % 8 for j in range(4)]

    def gather_start(uu, buf):
        pltpu.async_copy(table_hbm.at[idx_all.at[uu]], gbuf[buf], gsem[buf])

    def gather_wait(uu, buf):
        pltpu.make_async_copy(table_hbm.at[idx_all.at[uu]], gbuf[buf],
                              gsem[buf]).wait()

    def write_start(uu, buf):
        U = ubase + uu
        l = U // NQ
        q = U % NQ
        pltpu.async_copy(pbuf[buf].at[:, :, pl.ds(0, QB)],
                         out_hbm.at[l, :, q], wsem[buf])

    def write_wait(uu, buf):
        U = ubase + uu
        l = U // NQ
        q = U % NQ
        pltpu.make_async_copy(pbuf[buf].at[:, :, pl.ds(0, QB)],
                              out_hbm.at[l, :, q], wsem[buf]).wait()

    def transform(uu, buf):
        U = ubase + uu
        l = U // NQ
        G = gbuf[buf]
        P = pbuf[buf]
        posr = [pos_v[l, pl.ds(16 * j, 16)] for j in range(4)]

        @plsc.parallel_loop(0, QB, unroll=2)
        def _(t):
            t16 = jnp.full((16,), t, jnp.int32)
            for j in range(4):
                v = G[t, pl.ds(16 * j, 16)] + posr[j]
                plsc.store_scatter(P, [ehj[j], elj[j], t16], v)

    gather_start(0, 0)
    gather_start(1, 1)

    def outer(cc, _):
        for buf in range(2):
            uu = 2 * cc + buf
            gather_wait(uu, buf)

            @pl.when(uu >= 2)
            def _(uu=uu, buf=buf):
                write_wait(uu - 2, buf)

            transform(uu, buf)
            write_start(uu, buf)

            @pl.when(uu + 2 < UNITS_PER_W)
            def _(uu=uu, buf=buf):
                gather_start(uu + 2, buf)
        return 0

    lax.fori_loop(0, UNITS_PER_W // 2, outer, 0)

    write_wait(UNITS_PER_W - 2, 0)
    write_wait(UNITS_PER_W - 1, 1)


def kernel(seq, token_table, pos_table):
    mesh = plsc.VectorSubcoreMesh(core_axis_name="c", subcore_axis_name="s")

    tT = token_table.T
    t2 = pl.pallas_call(
        _tc_transpose_body,
        out_shape=jax.ShapeDtypeStruct((VOCAB, 128), jnp.float32),
        grid=(TGRID,),
        in_specs=[pl.BlockSpec((EMBED, TBLK), lambda i: (0, i))],
        out_specs=pl.BlockSpec((TBLK, 128), lambda i: (i, 0)),
    )(tT)

    seqT = seq.T.reshape(UNITS, QB)
    table = t2
    k5 = pl.kernel(
        _gather_body,
        out_type=jax.ShapeDtypeStruct((SEQ_LEN, 8, NQ, 8, QB), jnp.float32),
        mesh=mesh,
        scratch_types=[
            pltpu.VMEM((UNITS_PER_W, QB), jnp.int32),
            pltpu.VMEM((MAX_LEN, EMBED), jnp.float32),
            pltpu.VMEM((QB, 2 * EMBED), jnp.float32),
            pltpu.VMEM((QB, 2 * EMBED), jnp.float32),
            pltpu.VMEM((8, 8, QB + 1), jnp.float32),
            pltpu.VMEM((8, 8, QB + 1), jnp.float32),
            pltpu.SemaphoreType.DMA,
            pltpu.SemaphoreType.DMA,
            pltpu.SemaphoreType.DMA,
            pltpu.SemaphoreType.DMA,
        ],
        compiler_params=pltpu.CompilerParams(use_tc_tiling_on_sc=False,
                                             needs_layout_passes=False,
                                             disable_bounds_checks=True),
    )(seqT, table, pos_table)
    return k5.transpose(2, 4, 0, 1, 3).reshape(BATCH, SEQ_LEN, EMBED)
